# baseline (speedup 1.0000x reference)
import functools

import jax
import jax.numpy as jnp
from jax import lax
from jax.experimental import pallas as pl
from jax.experimental.pallas import tpu as pltpu
from jax.experimental.pallas import tpu_sc as plsc

UN = 50000
DD = 32
EE = 800000
LL = 2

NS = 16
CH = 400
EPT = EE // NS
NCHUNK = EPT // CH
UNP = 50176
RPT = UNP // NS
ZR = 32
NP = 50176
HPT = NP // NS

NI = 4
NGA = 2
GRP = 4
NGRP = (NCHUNK - 1) // GRP
NB = 5

CHH = 2000
NCH_H = EPT // CHH
NGH = NCH_H // NB

_mesh = plsc.VectorSubcoreMesh(core_axis_name="c", subcore_axis_name="s")
_f32 = jnp.float32


@functools.partial(
    pl.kernel,
    out_type=jax.ShapeDtypeStruct((4 * NP,), _f32),
    mesh=_mesh,
    compiler_params=pltpu.CompilerParams(use_tc_tiling_on_sc=False),
    scratch_types=(
        [pltpu.VMEM((CHH,), jnp.int32) for _ in range(NB)]
        + [
            pltpu.VMEM((CHH,), _f32),
            pltpu.VMEM((HPT,), _f32),
            pltpu.VMEM_SHARED((NP,), _f32),
            pltpu.VMEM_SHARED((NP,), _f32),
        ]
        + [pltpu.SemaphoreType.DMA] * (2 * NB)
    ),
)
def _hist_kernel(uu_r, ii_r, b_u, b_i, out, *scr):
    idx_v = scr[0:NB]
    ones_v, zer_v, hacc0, hacc1 = scr[NB:NB + 4]
    semi = scr[NB + 4:2 * NB + 4]
    sems = scr[2 * NB + 4:3 * NB + 4]
    cid = lax.axis_index("c")
    sid = lax.axis_index("s")
    for i in range(CHH // 16):
        ones_v[pl.ds(i * 16, 16)] = jnp.ones((16,), _f32)

    def zinit(i, _):
        zer_v[pl.ds(i * 16, 16)] = jnp.zeros((16,), _f32)
        return 0

    lax.fori_loop(0, HPT // 16, zinit, 0)
    pltpu.sync_copy(zer_v, hacc0.at[pl.ds(sid * HPT, HPT)])
    pltpu.sync_copy(zer_v, hacc1.at[pl.ds(sid * HPT, HPT)])
    plsc.subcore_barrier()

    def hist_pass(rowsf, hacc):
        ebase = sid * EPT

        def fire_idx(c, s):
            pltpu.async_copy(rowsf.at[pl.ds(ebase + c * CHH, CHH)],
                             idx_v[s], semi[s])

        def wait_idx(c, s):
            pltpu.make_async_copy(rowsf.at[pl.ds(ebase + c * CHH, CHH)],
                                  idx_v[s], semi[s]).wait()

        def fire_scatter(s):
            pltpu.async_copy(ones_v, hacc.at[idx_v[s]], sems[s], add=True)

        def wait_scatter(s):
            pltpu.make_async_copy(ones_v, hacc.at[idx_v[s]], sems[s]).wait()

        fire_idx(0, 0)
        fire_idx(1, 1)

        def group(g, _):
            for b in range(NB):
                c = g * NB + b
                s2 = (b + 2) % NB
                wait_idx(c, b)
                fire_scatter(b)
                pl.when(c >= 3)(lambda s2=s2: wait_scatter(s2))
                pl.when(c + 2 < NCH_H)(
                    lambda c=c, s2=s2: fire_idx(c + 2, s2))
            return 0

        lax.fori_loop(0, NGH, group, 0)
        wait_scatter((NCH_H - 3) % NB)
        wait_scatter((NCH_H - 2) % NB)
        wait_scatter((NCH_H - 1) % NB)

    c0 = cid == 0
    pl.when(c0)(lambda: hist_pass(uu_r, hacc0))
    pl.when(jnp.logical_not(c0))(lambda: hist_pass(ii_r, hacc0))
    pl.when(c0)(lambda: hist_pass(b_u, hacc1))
    pl.when(jnp.logical_not(c0))(lambda: hist_pass(b_i, hacc1))

    plsc.subcore_barrier()
    for p, hacc in enumerate((hacc0, hacc1)):
        pltpu.sync_copy(hacc.at[pl.ds(sid * HPT, HPT)], zer_v)
        pltpu.sync_copy(
            zer_v,
            out.at[pl.ds(p * 2 * NP + cid * NP + sid * HPT, HPT)],
        )


@functools.partial(
    pl.kernel,
    out_type=tuple(jax.ShapeDtypeStruct((UNP, DD), _f32) for _ in range(4)),
    mesh=_mesh,
    compiler_params=pltpu.CompilerParams(use_tc_tiling_on_sc=False),
    scratch_types=(
        [pltpu.VMEM((CH,), jnp.int32) for _ in range(NI)]
        + [pltpu.VMEM((CH,), jnp.int32) for _ in range(NI)]
        + [pltpu.VMEM((CH, DD), _f32) for _ in range(NGA)]
        + [pltpu.VMEM((ZR, DD), _f32)]
        + [pltpu.VMEM_SHARED((UNP, DD), _f32)]
        + [pltpu.SemaphoreType.DMA] * (NI + 2 * NGA + 1)
    ),
)
def _spmm_kernel(uu_r, uu_c, ii_r, ii_c, b_u, b_i,
                 t_uu, t_ii, t_qi, t_qu,
                 o1u, o1i, o2u, o2i, *scr):
    rows_v = scr[0:NI]
    cols_v = scr[NI:2 * NI]
    gath_v = scr[2 * NI:2 * NI + NGA]
    zer_v, acc = scr[2 * NI + NGA:2 * NI + NGA + 2]
    base = 2 * NI + NGA + 2
    semi = scr[base:base + NI]
    semg = scr[base + NI:base + NI + NGA]
    sems = scr[base + NI + NGA:base + NI + 2 * NGA]
    semz = scr[base + NI + 2 * NGA]
    cid = lax.axis_index("c")
    sid = lax.axis_index("s")
    c0 = cid == 0
    c1 = jnp.logical_not(c0)

    z16 = jnp.zeros((16,), _f32)
    for r in range(ZR):
        zer_v[r, pl.ds(0, 16)] = z16
        zer_v[r, pl.ds(16, 16)] = z16

    NZ = RPT // ZR
    LAG = 16

    def _zfire(j):
        pltpu.async_copy(zer_v, acc.at[pl.ds(sid * RPT + j * ZR, ZR)], semz)

    def _zwait():
        pltpu.make_async_copy(zer_v, acc.at[pl.ds(sid * RPT, ZR)],
                              semz).wait()

    def zero_step(j, _):
        pl.when(j < NZ)(lambda: _zfire(j))
        pl.when(j >= LAG)(_zwait)
        return 0

    lax.fori_loop(0, NZ + LAG, zero_step, 0)
    plsc.subcore_barrier()

    def spmm_pass(rowsf, colsf, tab):
        ebase = sid * EPT

        def fire_idx(c, si):
            pltpu.async_copy(rowsf.at[pl.ds(ebase + c * CH, CH)],
                             rows_v[si], semi[si])
            pltpu.async_copy(colsf.at[pl.ds(ebase + c * CH, CH)],
                             cols_v[si], semi[si])

        def wait_idx(c, si):
            pltpu.make_async_copy(rowsf.at[pl.ds(ebase + c * CH, CH)],
                                  rows_v[si], semi[si]).wait()
            pltpu.make_async_copy(colsf.at[pl.ds(ebase + c * CH, CH)],
                                  cols_v[si], semi[si]).wait()

        def fire_gather(si, sg):
            pltpu.async_copy(tab.at[cols_v[si]], gath_v[sg], semg[sg])

        def wait_gather(si, sg):
            pltpu.make_async_copy(tab.at[cols_v[si]], gath_v[sg],
                                  semg[sg]).wait()

        def fire_scatter(si, sg):
            pltpu.async_copy(gath_v[sg], acc.at[rows_v[si]], sems[sg],
                             add=True)

        def wait_scatter(si, sg):
            pltpu.make_async_copy(gath_v[sg], acc.at[rows_v[si]],
                                  sems[sg]).wait()

        def chunk_step(c, b, guard):
            si = b % NI
            sg = b % NGA
            sip = (b - 1) % NI
            sgp = (b - 1) % NGA
            sin = (b + 1) % NI
            sgn = (b + 1) % NGA
            si2 = (b + 2) % NI
            wait_gather(si, sg)
            fire_scatter(si, sg)
            if guard:
                pl.when(c >= 1)(lambda: wait_scatter(sip, sgp))
                pl.when(c + 1 < NCHUNK)(lambda: wait_idx(c + 1, sin))
                pl.when(c + 1 < NCHUNK)(lambda: fire_gather(sin, sgn))
                pl.when(c + 2 < NCHUNK)(lambda: fire_idx(c + 2, si2))
            else:
                if c >= 1:
                    wait_scatter(sip, sgp)
                if c + 1 < NCHUNK:
                    wait_idx(c + 1, sin)
                    fire_gather(sin, sgn)
                if c + 2 < NCHUNK:
                    fire_idx(c + 2, si2)

        fire_idx(0, 0)
        fire_idx(1, 1)
        wait_idx(0, 0)
        fire_gather(0, 0)

        def group(g, _):
            for b in range(GRP):
                chunk_step(g * GRP + b, b, True)
            return 0

        lax.fori_loop(0, NGRP, group, 0)
        for c in range(NGRP * GRP, NCHUNK):
            chunk_step(c, c % GRP, False)
        wait_scatter((NCHUNK - 1) % NI, (NCHUNK - 1) % NGA)

    def copy_out(out):
        pltpu.sync_copy(acc.at[pl.ds(sid * RPT, RPT)],
                        out.at[pl.ds(sid * RPT, RPT)])

    pl.when(c0)(lambda: spmm_pass(uu_r, uu_c, t_uu))
    pl.when(c1)(lambda: spmm_pass(ii_r, ii_c, t_ii))
    plsc.subcore_barrier()
    pl.when(c0)(lambda: copy_out(o1u))
    pl.when(c1)(lambda: copy_out(o1i))
    plsc.subcore_barrier()
    pl.when(c0)(lambda: spmm_pass(b_u, b_i, t_qi))
    pl.when(c1)(lambda: spmm_pass(b_i, b_u, t_qu))
    plsc.subcore_barrier()
    pl.when(c0)(lambda: copy_out(o2u))
    pl.when(c1)(lambda: copy_out(o2i))


BLK = 2000
NBLK = UN // BLK


def _dinv(deg):
    return jnp.where(deg > 0, lax.rsqrt(jnp.maximum(deg, 1e-12)), 0.0)


def _l2n(x):
    nrm = jnp.sqrt(jnp.sum(x * x, axis=-1, keepdims=True))
    return x / jnp.maximum(nrm, 1e-12)


def _prep_body(ue, ie, wu, bu, wi, bi, huu, hii, hbu, hbi,
               tuu, tii, tqi, tqu, gu_o, gi_o):
    duu = _dinv(huu[...])
    dii = _dinv(hii[...])
    dbu = _dinv(hbu[...])
    dbi = _dinv(hbi[...])
    gu = ue[...] * jax.nn.softmax(ue[...] @ wu[...] + bu[...], axis=1)
    gi = ie[...] * jax.nn.softmax(ie[...] @ wi[...] + bi[...], axis=1)
    tuu[...] = duu * gu
    tii[...] = dii * gi
    tqi[...] = dbi * gi
    tqu[...] = dbu * gu
    gu_o[...] = gu
    gi_o[...] = gi


def _combine_body(last, o1u, o1i, o2u, o2i, huu, hii, hbu, hbi, up, ip,
                  *outs):
    duu = _dinv(huu[...])
    dii = _dinv(hii[...])
    dbu = _dinv(hbu[...])
    dbi = _dinv(hbi[...])
    ue = (duu * o1u[...] + dbu * (o2u[...] - o1u[...])) * 0.5
    ie = (dii * o1i[...] + dbi * (o2i[...] - o1i[...])) * 0.5
    ua = up[...] + _l2n(ue)
    ia = ip[...] + _l2n(ie)
    if last:
        ua_o, ia_o = outs
        ua_o[...] = ua
        ia_o[...] = ia
    else:
        tuu, tii, tqi, tqu, ua_o, ia_o = outs
        tuu[...] = duu * ue
        tii[...] = dii * ie
        tqi[...] = dbi * ie
        tqu[...] = dbu * ue
        ua_o[...] = ua
        ia_o[...] = ia


_row_spec = pl.BlockSpec((BLK, DD), lambda i: (i, 0))
_w_spec = pl.BlockSpec((DD, DD), lambda i: (0, 0))
_b_spec = pl.BlockSpec((1, DD), lambda i: (0, 0))
_c_spec = pl.BlockSpec((BLK, 1), lambda i: (i, 0))

_rowUN = jax.ShapeDtypeStruct((UN, DD), _f32)


def _prep_call(ue, ie, wu, bu, wi, bi, hs):
    return pl.pallas_call(
        _prep_body,
        grid=(NBLK,),
        in_specs=[_row_spec, _row_spec, _w_spec, _b_spec, _w_spec, _b_spec,
                  _c_spec, _c_spec, _c_spec, _c_spec],
        out_specs=[_row_spec] * 6,
        out_shape=[_rowUN] * 6,
    )(ue, ie, wu, bu, wi, bi, *hs)


def _combine_call(last, o1u, o1i, o2u, o2i, hs, up, ip):
    n_out = 2 if last else 6
    return pl.pallas_call(
        functools.partial(_combine_body, last),
        grid=(NBLK,),
        in_specs=[_row_spec] * 4 + [_c_spec] * 4 + [_row_spec] * 2,
        out_specs=[_row_spec] * n_out,
        out_shape=[_rowUN] * n_out,
    )(o1u, o1i, o2u, o2i, *hs, up, ip)


def kernel(user_emb, item_emb, gating_weightu, gating_weightub,
           gating_weighti, gating_weightib,
           uu_rows, uu_cols, ii_rows, ii_cols, ui_rows, ui_cols):
    b_u = ui_rows[:EE]
    b_i = ui_cols[:EE] - jnp.int32(UN)

    hflat = _hist_kernel(uu_rows, ii_rows, b_u, b_i)
    h4 = hflat.reshape(4, NP)
    hs = tuple(h4[k].reshape(NP, 1) for k in range(4))

    tuu, tii, tqi, tqu, ua, ia = _prep_call(
        user_emb, item_emb, gating_weightu, gating_weightub,
        gating_weighti, gating_weightib, hs)

    for layer in range(LL):
        o1u, o1i, o2u, o2i = _spmm_kernel(
            uu_rows, uu_cols, ii_rows, ii_cols, b_u, b_i,
            tuu, tii, tqi, tqu)
        if layer + 1 < LL:
            tuu, tii, tqi, tqu, ua, ia = _combine_call(
                False, o1u, o1i, o2u, o2i, hs, ua, ia)
        else:
            ua, ia = _combine_call(True, o1u, o1i, o2u, o2i, hs, ua, ia)
    return jnp.concatenate([ua, ia], axis=0)
